# Initial kernel scaffold; baseline (speedup 1.0000x reference)
import jax
import jax.numpy as jnp
from jax.experimental import pallas as pl


def kernel(pred, target):
    raise NotImplementedError("write your pallas kernel here")


# speedup vs baseline: 2.8121x; 2.8121x over previous
import jax
import jax.numpy as jnp
from jax.experimental import pallas as pl
from jax.experimental.pallas import tpu as pltpu

_V = 100000
_SMOOTH = 0.1
_EPS = _SMOOTH / (_V - 1)
_CONF = 1.0 - _SMOOTH
_TB = 256
_VB = 4096
_NT = 2048 // _TB
_NV = (_V + _VB - 1) // _VB


def _body(tgt_ref, pred_ref, out_ref, s_ref, sp_ref, g_ref, num_ref, den_ref):
    t = pl.program_id(0)
    v = pl.program_id(1)
    x = pred_ref[...]
    col = jax.lax.broadcasted_iota(jnp.int32, (_TB, _VB), 1) + v * _VB
    valid = col < _V
    tgt = tgt_ref[...]
    hit = col == tgt
    e = jnp.where(valid, jnp.exp(x), 0.0)
    xs = jnp.where(valid, x, 0.0)
    se = jnp.sum(e, axis=1, keepdims=True)
    sx = jnp.sum(xs, axis=1, keepdims=True)
    gx = jnp.sum(jnp.where(hit, x, 0.0), axis=1, keepdims=True)

    @pl.when(v == 0)
    def _():
        s_ref[...] = se
        sp_ref[...] = sx
        g_ref[...] = gx

    @pl.when(v > 0)
    def _():
        s_ref[...] += se
        sp_ref[...] += sx
        g_ref[...] += gx

    @pl.when(v == _NV - 1)
    def _():
        lse = jnp.log(s_ref[...])
        sum_logprob = sp_ref[...] - _V * lse
        logp_tgt = g_ref[...] - lse
        per_tok = -_EPS * sum_logprob - (_CONF - _EPS) * logp_tgt
        mask = (tgt != 0).astype(jnp.float32)
        bn = jnp.sum(per_tok * mask)
        bd = jnp.sum(mask)

        @pl.when(t == 0)
        def _():
            num_ref[0, 0] = bn
            den_ref[0, 0] = bd

        @pl.when(t > 0)
        def _():
            num_ref[0, 0] += bn
            den_ref[0, 0] += bd

        @pl.when(t == _NT - 1)
        def _():
            out_ref[0, 0] = num_ref[0, 0] / jnp.maximum(den_ref[0, 0], 1.0)


def kernel(pred, target):
    pred2 = pred.reshape(-1, pred.shape[-1])
    tgt = target.reshape(-1, 1).astype(jnp.int32)
    out = pl.pallas_call(
        _body,
        grid=(_NT, _NV),
        in_specs=[
            pl.BlockSpec((_TB, 1), lambda t, v: (t, 0)),
            pl.BlockSpec((_TB, _VB), lambda t, v: (t, v)),
        ],
        out_specs=pl.BlockSpec(memory_space=pltpu.SMEM),
        out_shape=jax.ShapeDtypeStruct((1, 1), jnp.float32),
        scratch_shapes=[
            pltpu.VMEM((_TB, 1), jnp.float32),
            pltpu.VMEM((_TB, 1), jnp.float32),
            pltpu.VMEM((_TB, 1), jnp.float32),
            pltpu.SMEM((1, 1), jnp.float32),
            pltpu.SMEM((1, 1), jnp.float32),
        ],
        compiler_params=pltpu.CompilerParams(
            dimension_semantics=("arbitrary", "arbitrary"),
        ),
    )(tgt, pred2)
    return out[0, 0]
